# Initial kernel scaffold; baseline (speedup 1.0000x reference)
import jax
import jax.numpy as jnp
from jax.experimental import pallas as pl


def kernel(inputs, edge_index, embedding_layer, W1, b1, W2, b2, W3, b3, W4, b4, W5, b5):
    raise NotImplementedError("write your pallas kernel here")


# speedup vs baseline: 3.7786x; 3.7786x over previous
import functools

import jax
import jax.numpy as jnp
from jax import lax
from jax.experimental import pallas as pl
from jax.experimental.pallas import tpu as pltpu
from jax.experimental.pallas import tpu_sc as plsc

N = 10000
D = 128
E = 320000

NC = 2
NS = 16
NW = NC * NS

CH = 128

EC_PER_W = 79
E_PAD = NW * EC_PER_W * CH
N_ACC = 10112
ROWS_PER_TILE = N_ACC // NS

OFF = N_ACC
N2 = 20352
P2 = 2 * OFF
DC_PER_W = 157
D_PAD = NW * DC_PER_W * CH
DROWS_PER_TILE = N2 // NS

_SC_MESH = plsc.VectorSubcoreMesh(core_axis_name="c", subcore_axis_name="s")
_SC_PARAMS = pltpu.CompilerParams(use_tc_tiling_on_sc=False)


@functools.partial(
    pl.kernel,
    out_type=jax.ShapeDtypeStruct((NC * N2, 16), jnp.float32),
    mesh=_SC_MESH,
    scratch_types=[
        pltpu.VMEM_SHARED((N2, 16), jnp.float32),
        pltpu.VMEM((CH,), jnp.int32),
        pltpu.VMEM((CH, 16), jnp.float32),
    ],
    compiler_params=_SC_PARAMS,
)
def _deg_call(idx_hbm, ones_hbm, zeros_hbm, out_hbm, acc, idx_v, ones_v):
    cid = lax.axis_index("c")
    sid = lax.axis_index("s")
    wid = sid * NC + cid
    r0 = sid * DROWS_PER_TILE
    pltpu.sync_copy(ones_hbm, ones_v)
    pltpu.sync_copy(zeros_hbm, acc.at[pl.ds(r0, DROWS_PER_TILE)])
    plsc.subcore_barrier()

    @pl.loop(0, DC_PER_W)
    def _(g):
        c = wid * DC_PER_W + g
        pltpu.sync_copy(idx_hbm.at[c], idx_v)
        pltpu.sync_copy(ones_v, acc.at[idx_v], add=True)

    plsc.subcore_barrier()
    pltpu.sync_copy(acc.at[pl.ds(r0, DROWS_PER_TILE)],
                    out_hbm.at[pl.ds(cid * N2 + r0, DROWS_PER_TILE)])


@functools.partial(
    pl.kernel,
    out_type=jax.ShapeDtypeStruct((NC * N_ACC, D), jnp.float32),
    mesh=_SC_MESH,
    scratch_types=[
        pltpu.VMEM_SHARED((N_ACC, D), jnp.float32),
        pltpu.VMEM((CH,), jnp.int32),
        pltpu.VMEM((CH,), jnp.int32),
        pltpu.VMEM((CH, D), jnp.float32),
        pltpu.SemaphoreType.DMA,
    ],
    compiler_params=_SC_PARAMS,
)
def _agg_call(h_hbm, src_hbm, dst_hbm, zeros_hbm, out_hbm, acc, srcv, dstv,
              rows, sem):
    cid = lax.axis_index("c")
    sid = lax.axis_index("s")
    wid = sid * NC + cid
    r0 = sid * ROWS_PER_TILE
    pltpu.sync_copy(zeros_hbm, acc.at[pl.ds(r0, ROWS_PER_TILE)])
    plsc.subcore_barrier()

    @pl.loop(0, EC_PER_W)
    def _(g):
        c = wid * EC_PER_W + g
        pltpu.sync_copy(src_hbm.at[c], srcv)
        pltpu.sync_copy(dst_hbm.at[c], dstv)
        pltpu.async_copy(h_hbm.at[srcv], rows, sem).wait()
        pltpu.sync_copy(rows, acc.at[dstv], add=True)

    plsc.subcore_barrier()
    pltpu.sync_copy(acc.at[pl.ds(r0, ROWS_PER_TILE)],
                    out_hbm.at[pl.ds(cid * N_ACC + r0, ROWS_PER_TILE)])


BLN = 2544
BLC = 1264


def _norm_body(p0_ref, p1_ref, out_ref):
    deg = p0_ref[...] + p1_ref[...]
    col = jnp.maximum(deg[:, 0:1], 1.0)
    out_ref[...] = jnp.broadcast_to(lax.rsqrt(col), (BLN, D))


_norm_call = pl.pallas_call(
    _norm_body,
    grid=(N2 // BLN,),
    in_specs=[
        pl.BlockSpec((BLN, 16), lambda i: (i, 0)),
        pl.BlockSpec((BLN, 16), lambda i: (i, 0)),
    ],
    out_specs=pl.BlockSpec((BLN, D), lambda i: (i, 0)),
    out_shape=jax.ShapeDtypeStruct((N2, D), jnp.float32),
)


def _scale_body(x_ref, ns_ref, out_ref):
    out_ref[...] = x_ref[...] * ns_ref[...]


_scale_call = pl.pallas_call(
    _scale_body,
    grid=(N_ACC // BLC,),
    in_specs=[
        pl.BlockSpec((BLC, D), lambda i: (i, 0)),
        pl.BlockSpec((BLC, D), lambda i: (i, 0)),
    ],
    out_specs=pl.BlockSpec((BLC, D), lambda i: (i, 0)),
    out_shape=jax.ShapeDtypeStruct((N_ACC, D), jnp.float32),
)


def _layer_body(relu, p0_ref, p1_ref, nd_ref, ns_ref, w_ref, b_ref, h_ref,
                hs_ref):
    agg = (p0_ref[...] + p1_ref[...]) * nd_ref[...]
    y = jnp.dot(agg, w_ref[...], preferred_element_type=jnp.float32)
    y = y + b_ref[0:1, :]
    if relu:
        y = jnp.maximum(y, 0.0)
    h_ref[...] = y
    hs_ref[...] = y * ns_ref[...]


def _make_layer_call(relu):
    return pl.pallas_call(
        functools.partial(_layer_body, relu),
        grid=(N_ACC // BLC,),
        in_specs=[
            pl.BlockSpec((BLC, D), lambda i: (i, 0)),
            pl.BlockSpec((BLC, D), lambda i: (i, 0)),
            pl.BlockSpec((BLC, D), lambda i: (i, 0)),
            pl.BlockSpec((BLC, D), lambda i: (i, 0)),
            pl.BlockSpec((D, D), lambda i: (0, 0)),
            pl.BlockSpec((8, D), lambda i: (0, 0)),
        ],
        out_specs=[
            pl.BlockSpec((BLC, D), lambda i: (i, 0)),
            pl.BlockSpec((BLC, D), lambda i: (i, 0)),
        ],
        out_shape=[
            jax.ShapeDtypeStruct((N_ACC, D), jnp.float32),
            jax.ShapeDtypeStruct((N_ACC, D), jnp.float32),
        ],
    )


_layer_relu = _make_layer_call(True)
_layer_last = _make_layer_call(False)


def kernel(inputs, edge_index, embedding_layer, W1, b1, W2, b2, W3, b3, W4,
           b4, W5, b5):
    src = edge_index[0].astype(jnp.int32)
    dst = edge_index[1].astype(jnp.int32)

    pad_e = jnp.full((E_PAD - E,), N, jnp.int32)
    src_p = jnp.concatenate([src, pad_e]).reshape(E_PAD // CH, CH)
    dst_p = jnp.concatenate([dst, pad_e]).reshape(E_PAD // CH, CH)
    didx = jnp.concatenate(
        [src, dst + OFF, jnp.full((D_PAD - 2 * E,), P2, jnp.int32)]
    ).reshape(D_PAD // CH, CH)
    ones16 = jnp.ones((CH, 16), jnp.float32)
    zeros16 = jnp.zeros((DROWS_PER_TILE, 16), jnp.float32)
    zerosd = jnp.zeros((ROWS_PER_TILE, D), jnp.float32)
    x_pad = jnp.concatenate(
        [inputs, jnp.zeros((N_ACC - N, D), jnp.float32)])

    degp = _deg_call(didx, ones16, zeros16)
    norm_b = _norm_call(degp[:N2], degp[N2:])
    ns_b = norm_b[:N_ACC]
    nd_b = norm_b[OFF:OFF + N_ACC]

    h_cur = _scale_call(x_pad, ns_b)
    hs = []
    weights = [(W1, b1), (W2, b2), (W3, b3), (W4, b4), (W5, b5)]
    for i, (w, b) in enumerate(weights):
        parts = _agg_call(h_cur, src_p, dst_p, zerosd)
        b8 = jnp.broadcast_to(b[None, :], (8, D))
        call = _layer_relu if i < 4 else _layer_last
        h_out, h_next = call(parts[:N_ACC], parts[N_ACC:], nd_b, ns_b, w, b8)
        hs.append(h_out[:N])
        h_cur = h_next

    h5 = hs[4]
    emb_idx = jnp.clip(jnp.asarray(embedding_layer, jnp.int32) - 1, 0, 4)
    emb = lax.switch(emb_idx, [lambda h=h: h for h in hs])
    return (h5, emb, inputs)
